# baseline (speedup 1.0000x reference)
import jax
import jax.numpy as jnp
from jax.experimental import pallas as pl
from jax.experimental.pallas import tpu as pltpu

_T, _D, _DO, _E = 4096, 1024, 1024, 8
_BT = 1024


def _moe_body(x_ref, wr_ref, br_ref, web_ref, be_ref, o_ref):
    x = x_ref[...]
    logits = jnp.dot(x, wr_ref[...], preferred_element_type=jnp.float32)
    logits = logits + br_ref[...]
    m = jnp.max(logits, axis=-1, keepdims=True)
    p = jnp.exp(logits - m)
    w = p / jnp.sum(p, axis=-1, keepdims=True)
    rank = jnp.zeros_like(w)
    for j in range(_E):
        rank = rank + (w[:, j : j + 1] > w).astype(jnp.float32)
    sw = jnp.where(rank < 2.0, w, 0.0)
    acc = jnp.dot(sw, be_ref[...], preferred_element_type=jnp.float32)
    for e in range(0, _E, 2):
        xcat = jnp.concatenate(
            [(x * sw[:, e : e + 1]).astype(jnp.bfloat16),
             (x * sw[:, e + 1 : e + 2]).astype(jnp.bfloat16)], axis=1)
        y = jnp.dot(xcat, web_ref[e // 2],
                    preferred_element_type=jnp.float32)
        acc = acc + y
    o_ref[...] = acc


def kernel(x, Wr, br, We, be):
    br2 = br.reshape(1, _E)
    web = We.astype(jnp.bfloat16).reshape(_E // 2, 2 * _D, _DO)
    return pl.pallas_call(
        _moe_body,
        grid=(_T // _BT,),
        in_specs=[
            pl.BlockSpec((_BT, _D), lambda i: (i, 0)),
            pl.BlockSpec((_D, _E), lambda i: (0, 0)),
            pl.BlockSpec((1, _E), lambda i: (0, 0)),
            pl.BlockSpec((_E // 2, 2 * _D, _DO), lambda i: (0, 0, 0)),
            pl.BlockSpec((_E, _DO), lambda i: (0, 0)),
        ],
        out_specs=pl.BlockSpec((_BT, _DO), lambda i: (i, 0)),
        out_shape=jax.ShapeDtypeStruct((_T, _DO), jnp.float32),
        compiler_params=pltpu.CompilerParams(
            dimension_semantics=("arbitrary",),
        ),
    )(x, Wr, br2, web, be)


# speedup vs baseline: 1.1861x; 1.1861x over previous
import jax
import jax.numpy as jnp
from jax.experimental import pallas as pl
from jax.experimental.pallas import tpu as pltpu

_T, _D, _DO, _E = 4096, 1024, 1024, 8
_BT = 256


def _moe_body(x_ref, wr_ref, br_ref, we_ref, be_ref, o_ref, web_ref):
    @pl.when(pl.program_id(0) == 0)
    def _cast_weights():
        for e in range(_E):
            web_ref[pl.ds(e * _D, _D), :] = we_ref[e].astype(jnp.bfloat16)

    x = x_ref[...]
    logits = jnp.dot(x, wr_ref[...], preferred_element_type=jnp.float32)
    logits = logits + br_ref[...]
    m = jnp.max(logits, axis=-1, keepdims=True)
    p = jnp.exp(logits - m)
    w = p / jnp.sum(p, axis=-1, keepdims=True)
    rank = jnp.zeros_like(w)
    for j in range(_E):
        rank = rank + (w[:, j : j + 1] > w).astype(jnp.float32)
    sw = jnp.where(rank < 2.0, w, 0.0)
    acc = jnp.dot(sw, be_ref[...], preferred_element_type=jnp.float32)
    xb = x.astype(jnp.bfloat16)
    for e in range(_E):
        y = jnp.dot(xb, web_ref[pl.ds(e * _D, _D), :],
                    preferred_element_type=jnp.float32)
        acc = acc + sw[:, e : e + 1] * y
    o_ref[...] = acc


def kernel(x, Wr, br, We, be):
    br2 = br.reshape(1, _E)
    return pl.pallas_call(
        _moe_body,
        grid=(_T // _BT,),
        in_specs=[
            pl.BlockSpec((_BT, _D), lambda i: (i, 0)),
            pl.BlockSpec((_D, _E), lambda i: (0, 0)),
            pl.BlockSpec((1, _E), lambda i: (0, 0)),
            pl.BlockSpec((_E, _D, _DO), lambda i: (0, 0, 0)),
            pl.BlockSpec((_E, _DO), lambda i: (0, 0)),
        ],
        out_specs=pl.BlockSpec((_BT, _DO), lambda i: (i, 0)),
        out_shape=jax.ShapeDtypeStruct((_T, _DO), jnp.float32),
        scratch_shapes=[pltpu.VMEM((_E * _D, _DO), jnp.bfloat16)],
        compiler_params=pltpu.CompilerParams(
            dimension_semantics=("arbitrary",),
        ),
    )(x, Wr, br2, We, be)
